# Initial kernel scaffold; baseline (speedup 1.0000x reference)
import jax
import jax.numpy as jnp
from jax.experimental import pallas as pl


def kernel(agent_idx, task_idx, theta, beta):
    raise NotImplementedError("write your pallas kernel here")


# speedup vs baseline: 1.1166x; 1.1166x over previous
import functools

import jax
import jax.numpy as jnp
from jax import lax
from jax.experimental import pallas as pl
from jax.experimental.pallas import tpu as pltpu
from jax.experimental.pallas import tpu_sc as plsc

NUM_WORKERS = 32
BATCH_SIZE = 16384
PER_WORKER = BATCH_SIZE // NUM_WORKERS
CHUNK = 128
NUM_CHUNKS = PER_WORKER // CHUNK
LANES = 16

_mesh = plsc.VectorSubcoreMesh(core_axis_name="c", subcore_axis_name="s")


@functools.partial(
    pl.kernel,
    mesh=_mesh,
    out_type=jax.ShapeDtypeStruct((BATCH_SIZE,), jnp.float32),
    scratch_types=[
        pltpu.VMEM((PER_WORKER,), jnp.int32),
        pltpu.VMEM((PER_WORKER,), jnp.int32),
        pltpu.VMEM((PER_WORKER,), jnp.float32),
        pltpu.VMEM((PER_WORKER,), jnp.float32),
        pltpu.VMEM((PER_WORKER,), jnp.float32),
        pltpu.SemaphoreType.DMA,
    ],
)
def _irt_sc_kernel(agent_idx_hbm, task_idx_hbm, theta_hbm, beta_hbm,
                   out_hbm, aidx_v, tidx_v, th_v, be_v, o_v, sem):
    wid = lax.axis_index("s") * 2 + lax.axis_index("c")
    base = wid * PER_WORKER
    pltpu.sync_copy(agent_idx_hbm.at[pl.ds(base, PER_WORKER)], aidx_v)
    pltpu.sync_copy(task_idx_hbm.at[pl.ds(base, PER_WORKER)], tidx_v)
    copies = []
    for j in range(NUM_CHUNKS):
        sl = pl.ds(j * CHUNK, CHUNK)
        copies.append(
            pltpu.async_copy(theta_hbm.at[aidx_v.at[sl]], th_v.at[sl], sem))
        copies.append(
            pltpu.async_copy(beta_hbm.at[tidx_v.at[sl]], be_v.at[sl], sem))
    for c in copies:
        c.wait()
    for i in range(PER_WORKER // LANES):
        sl = pl.ds(i * LANES, LANES)
        o_v[sl] = th_v[sl] - be_v[sl]
    pltpu.sync_copy(o_v, out_hbm.at[pl.ds(base, PER_WORKER)])


def kernel(agent_idx, task_idx, theta, beta):
    return _irt_sc_kernel(
        agent_idx.astype(jnp.int32),
        task_idx.astype(jnp.int32),
        theta.reshape(-1),
        beta.reshape(-1),
    )


# speedup vs baseline: 3.3077x; 2.9622x over previous
import functools

import jax
import jax.numpy as jnp
from jax import lax
from jax.experimental import pallas as pl
from jax.experimental.pallas import tpu as pltpu
from jax.experimental.pallas import tpu_sc as plsc

NUM_WORKERS = 32
BATCH_SIZE = 16384
PER_WORKER = BATCH_SIZE // NUM_WORKERS
CHUNK = 128
NUM_CHUNKS = PER_WORKER // CHUNK
LANES = 16

_mesh = plsc.VectorSubcoreMesh(core_axis_name="c", subcore_axis_name="s")


@functools.partial(
    pl.kernel,
    mesh=_mesh,
    out_type=jax.ShapeDtypeStruct((BATCH_SIZE,), jnp.float32),
    scratch_types=[
        pltpu.VMEM((1, PER_WORKER), jnp.int32),
        pltpu.VMEM((1, PER_WORKER), jnp.int32),
        pltpu.VMEM((1, PER_WORKER), jnp.float32),
        pltpu.VMEM((1, PER_WORKER), jnp.float32),
        pltpu.VMEM((PER_WORKER,), jnp.float32),
        pltpu.SemaphoreType.DMA,
    ],
)
def _irt_sc_kernel(agent_idx_hbm, task_idx_hbm, theta_hbm, beta_hbm,
                   out_hbm, aidx_v, tidx_v, th_v, be_v, o_v, sem):
    wid = lax.axis_index("s") * 2 + lax.axis_index("c")
    base = wid * PER_WORKER
    pltpu.sync_copy(agent_idx_hbm.at[pl.ds(base, PER_WORKER)], aidx_v.at[0])
    pltpu.sync_copy(task_idx_hbm.at[pl.ds(base, PER_WORKER)], tidx_v.at[0])
    copies = []
    for j in range(NUM_CHUNKS):
        sl = pl.ds(j * CHUNK, CHUNK)
        copies.append(pltpu.async_copy(
            theta_hbm.at[aidx_v.at[:, sl]], th_v.at[:, sl], sem))
        copies.append(pltpu.async_copy(
            beta_hbm.at[tidx_v.at[:, sl]], be_v.at[:, sl], sem))
    for c in copies:
        c.wait()
    for i in range(PER_WORKER // LANES):
        sl = pl.ds(i * LANES, LANES)
        o_v[sl] = th_v[0, sl] - be_v[0, sl]
    pltpu.sync_copy(o_v, out_hbm.at[pl.ds(base, PER_WORKER)])


def kernel(agent_idx, task_idx, theta, beta):
    return _irt_sc_kernel(
        agent_idx.astype(jnp.int32),
        task_idx.astype(jnp.int32),
        theta.T,
        beta.T,
    )


# speedup vs baseline: 3.3234x; 1.0047x over previous
import functools

import jax
import jax.numpy as jnp
from jax import lax
from jax.experimental import pallas as pl
from jax.experimental.pallas import tpu as pltpu
from jax.experimental.pallas import tpu_sc as plsc

NUM_WORKERS = 32
BATCH_SIZE = 16384
PER_WORKER = BATCH_SIZE // NUM_WORKERS
CHUNK = 512
NUM_CHUNKS = PER_WORKER // CHUNK
LANES = 16

_mesh = plsc.VectorSubcoreMesh(core_axis_name="c", subcore_axis_name="s")


@functools.partial(
    pl.kernel,
    mesh=_mesh,
    out_type=jax.ShapeDtypeStruct((BATCH_SIZE,), jnp.float32),
    scratch_types=[
        pltpu.VMEM((1, PER_WORKER), jnp.int32),
        pltpu.VMEM((1, PER_WORKER), jnp.int32),
        pltpu.VMEM((1, PER_WORKER), jnp.float32),
        pltpu.VMEM((1, PER_WORKER), jnp.float32),
        pltpu.VMEM((PER_WORKER,), jnp.float32),
        pltpu.SemaphoreType.DMA,
    ],
)
def _irt_sc_kernel(agent_idx_hbm, task_idx_hbm, theta_hbm, beta_hbm,
                   out_hbm, aidx_v, tidx_v, th_v, be_v, o_v, sem):
    wid = lax.axis_index("s") * 2 + lax.axis_index("c")
    base = wid * PER_WORKER
    pltpu.sync_copy(agent_idx_hbm.at[pl.ds(base, PER_WORKER)], aidx_v.at[0])
    pltpu.sync_copy(task_idx_hbm.at[pl.ds(base, PER_WORKER)], tidx_v.at[0])
    copies = []
    for j in range(NUM_CHUNKS):
        sl = pl.ds(j * CHUNK, CHUNK)
        copies.append(pltpu.async_copy(
            theta_hbm.at[aidx_v.at[:, sl]], th_v.at[:, sl], sem))
        copies.append(pltpu.async_copy(
            beta_hbm.at[tidx_v.at[:, sl]], be_v.at[:, sl], sem))
    for c in copies:
        c.wait()
    for i in range(PER_WORKER // LANES):
        sl = pl.ds(i * LANES, LANES)
        o_v[sl] = th_v[0, sl] - be_v[0, sl]
    pltpu.sync_copy(o_v, out_hbm.at[pl.ds(base, PER_WORKER)])


def kernel(agent_idx, task_idx, theta, beta):
    return _irt_sc_kernel(
        agent_idx.astype(jnp.int32),
        task_idx.astype(jnp.int32),
        theta.T,
        beta.T,
    )
